# baseline (speedup 1.0000x reference)
import functools

import jax
import jax.numpy as jnp
from jax import lax
from jax.experimental import pallas as pl
from jax.experimental.pallas import tpu as pltpu
from jax.experimental.pallas import tpu_sc as plsc

SIDE = 256
SCALE = 3.0
NPTS = 1048576
NVOX = SIDE * SIDE * SIDE

NC = 2
NS = 16
NW = NC * NS
LANES = 16

CP = 128
PTS_PER_TILE = NPTS // NW
N_CHUNKS = PTS_PER_TILE // CP

ZY = SIDE * SIDE
ZY_PER_TILE = ZY // NW
BROWS = 16
ROWF = 4 * SIDE


def _build_body(src, table, in_v, out_v, pat_s, sem):
    wid = lax.axis_index("s") * NC + lax.axis_index("c")
    row_base = wid * ZY_PER_TILE

    iota = lax.iota(jnp.int32, LANES)
    cpat = lax.bitwise_and(lax.shift_right_logical(iota, 1), 3)
    xpat = (lax.shift_left(lax.shift_right_logical(iota, 3), 1)
            + lax.bitwise_and(iota, 1))
    pat_s[pl.ds(0, LANES)] = cpat
    pat_s[pl.ds(LANES, LANES)] = xpat

    def chunk(i, carry):
        r0 = row_base + i * BROWS
        z = r0 // SIDE
        y0 = r0 % SIDE
        pltpu.sync_copy(src.at[0, :, z, pl.ds(y0, BROWS), :], in_v)
        cp = pat_s[pl.ds(0, LANES)]
        xp = pat_s[pl.ds(LANES, LANES)]
        for b in range(BROWS):
            yi = jnp.full((LANES,), b, jnp.int32)
            for j in range(ROWF // LANES):
                out_v[pl.ds(b * ROWF + j * LANES, LANES)] = (
                    plsc.load_gather(in_v, [cp, yi, xp + 4 * j]))
        pltpu.sync_copy(out_v, table.at[pl.ds(r0 * ROWF, BROWS * ROWF)])
        return carry

    lax.fori_loop(0, ZY_PER_TILE // BROWS, chunk, 0, unroll=False)


def _build_table(voxels):
    mesh = plsc.VectorSubcoreMesh(
        core_axis_name="c", subcore_axis_name="s",
        num_cores=NC, num_subcores=NS)
    f = pl.kernel(
        _build_body,
        out_type=jax.ShapeDtypeStruct((NVOX * 4,), jnp.float32),
        mesh=mesh,
        scratch_types=[
            pltpu.VMEM((4, BROWS, SIDE), jnp.float32),
            pltpu.VMEM((BROWS * ROWF,), jnp.float32),
            pltpu.VMEM((2 * LANES,), jnp.int32),
            pltpu.SemaphoreType.DMA,
        ],
        compiler_params=pltpu.CompilerParams(
            needs_layout_passes=False, use_tc_tiling_on_sc=False),
    )
    return f(voxels)


def _sc_body(pos, table, biasv, out,
             pos_b, idx_b, w_b, par_b, dst_b, outs_b, dummy_v, bias_v,
             sem_pos, sem_g, sem_out):
    wid = lax.axis_index("s") * NC + lax.axis_index("c")
    tile_base = wid * PTS_PER_TILE

    pltpu.sync_copy(biasv, bias_v)
    iota = lax.iota(jnp.int32, LANES)
    colsel2 = lax.rem(iota, 4) * 2
    rowexp0 = lax.div(iota, 4)
    bvec = bias_v[...]

    def pos_start(i, b):
        base = tile_base + i * CP
        pltpu.async_copy(pos.at[pl.ds(base, CP), :], pos_b.at[b], sem_pos)

    def pos_wait():
        pltpu.make_async_copy(
            pos.at[pl.ds(tile_base, CP), :], pos_b.at[0], sem_pos).wait()

    def compute(b):
        for g in range(CP // LANES):
            rows = iota + (g * LANES)
            x = plsc.load_gather(pos_b.at[b], [rows, iota * 0])
            y = plsc.load_gather(pos_b.at[b], [rows, iota * 0 + 1])
            z = plsc.load_gather(pos_b.at[b], [rows, iota * 0 + 2])

            def axis(c):
                r = c / SCALE
                i_f = ((r + 1.0) * SIDE - 1.0) * 0.5
                i_f = jnp.minimum(jnp.maximum(i_f, 0.0), float(SIDE - 1))
                i0 = i_f.astype(jnp.int32)
                t = i_f - i0.astype(jnp.float32)
                return i0, t

            x0, tx = axis(x)
            y0, ty = axis(y)
            z0, tz = axis(z)
            dx = jnp.where(x0 < SIDE - 1, 1, 0)
            x1 = x0 + dx
            xh0 = lax.shift_right_logical(x0, 1)
            xh1 = lax.shift_right_logical(x1, 1)
            par0 = lax.bitwise_and(x0, 1)
            par1 = lax.bitwise_and(x1, 1)
            dy2 = jnp.where(y0 < SIDE - 1, SIDE // 2, 0)
            dz2 = jnp.where(z0 < SIDE - 1, (SIDE * SIDE) // 2, 0)
            zyb = (z0 * SIDE + y0) * (SIDE // 2)
            wx0 = 1.0 - tx
            wy0 = 1.0 - ty
            wz0 = 1.0 - tz

            zybs = (zyb, zyb + dy2, zyb + dz2, zyb + dz2 + dy2)
            azys = (wz0 * wy0, wz0 * ty, tz * wy0, tz * ty)
            off = g * LANES
            par_b[b, pl.ds(off, LANES)] = par0
            par_b[b, pl.ds(CP + off, LANES)] = par1
            for zy in range(4):
                for side in range(2):
                    gi = zy * 2 + side
                    idx_b[b, pl.ds(gi * CP + off, LANES)] = (
                        zybs[zy] + (xh1 if side else xh0))
                    w_b[b, pl.ds(gi * CP + off, LANES)] = (
                        azys[zy] * (tx if side else wx0))

    def fire(b):
        for gi in range(8):
            pltpu.async_copy(
                table.at[idx_b.at[b, pl.ds(gi * CP, CP)]],
                dst_b.at[b, pl.ds(gi * CP, CP)],
                sem_g,
            )

    def gather_drain(b):
        pltpu.make_async_copy(
            table.at[pl.ds(0, 8 * CP)], dst_b.at[b], sem_g).wait()

    def out_wait():
        pltpu.make_async_copy(
            outs_b.at[0], out.at[pl.ds(tile_base * 4, CP * 4)],
            sem_out).wait()

    def combine(i, b):
        for j in range(CP // 4):
            rowe = rowexp0 + (4 * j)
            col0 = colsel2 + plsc.load_gather(par_b.at[b], [rowe])
            col1 = colsel2 + plsc.load_gather(par_b.at[b], [rowe + CP])
            acc = bvec
            for zy in range(4):
                for side in range(2):
                    gi = zy * 2 + side
                    r = rowe + (gi * CP)
                    w = plsc.load_gather(w_b.at[b], [r])
                    v = plsc.load_gather(
                        dst_b.at[b], [r, col1 if side else col0])
                    acc = acc + w * v
            outs_b[b, pl.ds(j * LANES, LANES)] = acc
        base = tile_base + i * CP
        pltpu.async_copy(outs_b.at[b],
                         out.at[pl.ds(base * 4, CP * 4)], sem_out)

    pltpu.async_copy(pos.at[pl.ds(tile_base, CP), :], pos_b.at[0], sem_pos)
    pltpu.async_copy(out.at[pl.ds(tile_base * 4, CP * 4)], dummy_v, sem_out)
    pltpu.async_copy(out.at[pl.ds(tile_base * 4, CP * 4)], dummy_v, sem_out)
    pos_wait()
    compute(0)
    fire(0)
    pos_start(1, 1)

    def step(k, carry):
        for b in range(2):
            i = 2 * k + b
            pos_wait()
            compute(1 - b)
            gather_drain(b)
            fire(1 - b)
            pos_start(jnp.minimum(i + 2, N_CHUNKS - 1), b)
            out_wait()
            combine(i, b)
        return carry

    lax.fori_loop(0, N_CHUNKS // 2, step, 0, unroll=False)

    pos_wait()
    gather_drain(0)
    out_wait()
    out_wait()


@jax.jit
def _voxel_sample(pos, table, biasv):
    mesh = plsc.VectorSubcoreMesh(
        core_axis_name="c", subcore_axis_name="s",
        num_cores=NC, num_subcores=NS)
    f = pl.kernel(
        functools.partial(_sc_body),
        out_type=jax.ShapeDtypeStruct((NPTS * 4,), jnp.float32),
        mesh=mesh,
        scratch_types=[
            pltpu.VMEM((2, CP, 3), jnp.float32),
            pltpu.VMEM((2, 8 * CP), jnp.int32),
            pltpu.VMEM((2, 8 * CP), jnp.float32),
            pltpu.VMEM((2, 2 * CP), jnp.int32),
            pltpu.VMEM((2, 8 * CP, 8), jnp.float32),
            pltpu.VMEM((2, CP * 4), jnp.float32),
            pltpu.VMEM((CP * 4,), jnp.float32),
            pltpu.VMEM((LANES,), jnp.float32),
            pltpu.SemaphoreType.DMA,
            pltpu.SemaphoreType.DMA,
            pltpu.SemaphoreType.DMA,
        ],
        compiler_params=pltpu.CompilerParams(
            needs_layout_passes=False, use_tc_tiling_on_sc=False),
    )
    return f(pos, table, biasv)


def kernel(positions, voxels, bias):
    table = _build_table(voxels).reshape(NVOX // 2, 8)
    biasv = jnp.tile(bias[0], 4)
    flat_out = _voxel_sample(positions, table, biasv)
    return flat_out.reshape(NPTS, 4)


# speedup vs baseline: 1.5163x; 1.5163x over previous
import functools

import jax
import jax.numpy as jnp
from jax import lax
from jax.experimental import pallas as pl
from jax.experimental.pallas import tpu as pltpu
from jax.experimental.pallas import tpu_sc as plsc

SIDE = 256
SCALE = 3.0
NPTS = 1048576
NVOX = SIDE * SIDE * SIDE

NC = 2
NS = 16
NW = NC * NS
LANES = 16

CP = 128
PTS_PER_TILE = NPTS // NW
N_CHUNKS = PTS_PER_TILE // CP

ZY = SIDE * SIDE
ZY_PER_TILE = ZY // NW
BROWS = 16
ROWF = 4 * SIDE


def _build_body(src, table, in_v, out_v, dummy_v, pat_s, sem_in, sem_out):
    wid = lax.axis_index("s") * NC + lax.axis_index("c")
    row_base = wid * ZY_PER_TILE
    n_bchunks = ZY_PER_TILE // BROWS

    iota = lax.iota(jnp.int32, LANES)
    cpat = lax.bitwise_and(lax.shift_right_logical(iota, 1), 3)
    xpat = (lax.shift_left(lax.shift_right_logical(iota, 3), 1)
            + lax.bitwise_and(iota, 1))
    pat_s[pl.ds(0, LANES)] = cpat
    pat_s[pl.ds(LANES, LANES)] = xpat

    def in_start(i, b):
        r0 = row_base + i * BROWS
        z = r0 // SIDE
        y0 = lax.rem(r0, SIDE)
        pltpu.async_copy(src.at[0, :, z, pl.ds(y0, BROWS), :],
                         in_v.at[b], sem_in)

    def in_wait():
        pltpu.make_async_copy(
            src.at[0, :, 0, pl.ds(0, BROWS), :], in_v.at[0], sem_in).wait()

    def out_wait():
        pltpu.make_async_copy(
            out_v.at[0], table.at[pl.ds(0, BROWS * ROWF)], sem_out).wait()

    def interleave(i, b):
        cp = pat_s[pl.ds(0, LANES)]
        xp = pat_s[pl.ds(LANES, LANES)]
        for r in range(BROWS):
            yi = jnp.full((LANES,), r, jnp.int32)
            for j in range(ROWF // LANES):
                out_v[b, pl.ds(r * ROWF + j * LANES, LANES)] = (
                    plsc.load_gather(in_v.at[b], [cp, yi, xp + 4 * j]))
        r0 = row_base + i * BROWS
        pltpu.async_copy(out_v.at[b],
                         table.at[pl.ds(r0 * ROWF, BROWS * ROWF)], sem_out)

    in_start(0, 0)
    pltpu.async_copy(table.at[pl.ds(0, BROWS * ROWF)], dummy_v, sem_out)
    pltpu.async_copy(table.at[pl.ds(0, BROWS * ROWF)], dummy_v, sem_out)
    in_start(1, 1)

    def step(k, carry):
        for b in range(2):
            i = 2 * k + b
            in_wait()
            in_start(jnp.minimum(i + 2, n_bchunks - 1), b)
            out_wait()
            interleave(i, b)
        return carry

    lax.fori_loop(0, n_bchunks // 2, step, 0, unroll=False)

    in_wait()
    in_wait()
    out_wait()
    out_wait()


def _build_table(voxels):
    mesh = plsc.VectorSubcoreMesh(
        core_axis_name="c", subcore_axis_name="s",
        num_cores=NC, num_subcores=NS)
    f = pl.kernel(
        _build_body,
        out_type=jax.ShapeDtypeStruct((NVOX * 4,), jnp.float32),
        mesh=mesh,
        scratch_types=[
            pltpu.VMEM((2, 4, BROWS, SIDE), jnp.float32),
            pltpu.VMEM((2, BROWS * ROWF), jnp.float32),
            pltpu.VMEM((BROWS * ROWF,), jnp.float32),
            pltpu.VMEM((2 * LANES,), jnp.int32),
            pltpu.SemaphoreType.DMA,
            pltpu.SemaphoreType.DMA,
        ],
        compiler_params=pltpu.CompilerParams(
            needs_layout_passes=False, use_tc_tiling_on_sc=False),
    )
    return f(voxels)


def _sc_body(xs, ys, zs, table, biasv, out,
             pos_b, idx_b, w_b, par_b, dst_b, outs_b, dummy_v, bias_v,
             sem_pos, sem_g, sem_out):
    wid = lax.axis_index("s") * NC + lax.axis_index("c")
    tile_base = wid * PTS_PER_TILE

    pltpu.sync_copy(biasv, bias_v)
    iota = lax.iota(jnp.int32, LANES)
    colsel2 = lax.rem(iota, 4) * 2
    rowexp0 = lax.div(iota, 4)
    bvec = bias_v[...]

    def pos_start(i, b):
        base = tile_base + i * CP
        pltpu.async_copy(xs.at[pl.ds(base, CP)], pos_b.at[b, 0], sem_pos)
        pltpu.async_copy(ys.at[pl.ds(base, CP)], pos_b.at[b, 1], sem_pos)
        pltpu.async_copy(zs.at[pl.ds(base, CP)], pos_b.at[b, 2], sem_pos)

    def pos_wait():
        for a in range(3):
            pltpu.make_async_copy(
                xs.at[pl.ds(tile_base, CP)], pos_b.at[0, a], sem_pos).wait()

    def compute(b):
        for g in range(CP // LANES):
            sl = pl.ds(g * LANES, LANES)
            x = pos_b[b, 0, sl]
            y = pos_b[b, 1, sl]
            z = pos_b[b, 2, sl]

            def axis(c):
                r = c / SCALE
                i_f = ((r + 1.0) * SIDE - 1.0) * 0.5
                i_f = jnp.minimum(jnp.maximum(i_f, 0.0), float(SIDE - 1))
                i0 = i_f.astype(jnp.int32)
                t = i_f - i0.astype(jnp.float32)
                return i0, t

            x0, tx = axis(x)
            y0, ty = axis(y)
            z0, tz = axis(z)
            dx = jnp.where(x0 < SIDE - 1, 1, 0)
            x1 = x0 + dx
            xh0 = lax.shift_right_logical(x0, 1)
            xh1 = lax.shift_right_logical(x1, 1)
            par0 = lax.bitwise_and(x0, 1)
            par1 = lax.bitwise_and(x1, 1)
            dy2 = jnp.where(y0 < SIDE - 1, SIDE // 2, 0)
            dz2 = jnp.where(z0 < SIDE - 1, (SIDE * SIDE) // 2, 0)
            zyb = (z0 * SIDE + y0) * (SIDE // 2)
            wx0 = 1.0 - tx
            wy0 = 1.0 - ty
            wz0 = 1.0 - tz

            zybs = (zyb, zyb + dy2, zyb + dz2, zyb + dz2 + dy2)
            azys = (wz0 * wy0, wz0 * ty, tz * wy0, tz * ty)
            off = g * LANES
            par_b[b, pl.ds(off, LANES)] = par0
            par_b[b, pl.ds(CP + off, LANES)] = par1
            for zy in range(4):
                for side in range(2):
                    gi = zy * 2 + side
                    idx_b[b, pl.ds(gi * CP + off, LANES)] = (
                        zybs[zy] + (xh1 if side else xh0))
                    w_b[b, pl.ds(gi * CP + off, LANES)] = (
                        azys[zy] * (tx if side else wx0))

    def fire(b):
        for gi in range(8):
            pltpu.async_copy(
                table.at[idx_b.at[b, pl.ds(gi * CP, CP)]],
                dst_b.at[b, pl.ds(gi * CP, CP)],
                sem_g,
            )

    def gather_drain(b):
        pltpu.make_async_copy(
            table.at[pl.ds(0, 8 * CP)], dst_b.at[b], sem_g).wait()

    def out_wait():
        pltpu.make_async_copy(
            outs_b.at[0], out.at[pl.ds(tile_base * 4, CP * 4)],
            sem_out).wait()

    def combine(i, b):
        for j in range(CP // 4):
            rowe = rowexp0 + (4 * j)
            col0 = colsel2 + plsc.load_gather(par_b.at[b], [rowe])
            col1 = colsel2 + plsc.load_gather(par_b.at[b], [rowe + CP])
            acc = bvec
            for zy in range(4):
                for side in range(2):
                    gi = zy * 2 + side
                    r = rowe + (gi * CP)
                    w = plsc.load_gather(w_b.at[b], [r])
                    v = plsc.load_gather(
                        dst_b.at[b], [r, col1 if side else col0])
                    acc = acc + w * v
            outs_b[b, pl.ds(j * LANES, LANES)] = acc
        base = tile_base + i * CP
        pltpu.async_copy(outs_b.at[b],
                         out.at[pl.ds(base * 4, CP * 4)], sem_out)

    pos_start(0, 0)
    pltpu.async_copy(out.at[pl.ds(tile_base * 4, CP * 4)], dummy_v, sem_out)
    pltpu.async_copy(out.at[pl.ds(tile_base * 4, CP * 4)], dummy_v, sem_out)
    pos_wait()
    compute(0)
    fire(0)
    pos_start(1, 1)

    def step(k, carry):
        for b in range(2):
            i = 2 * k + b
            pos_wait()
            compute(1 - b)
            gather_drain(b)
            fire(1 - b)
            pos_start(jnp.minimum(i + 2, N_CHUNKS - 1), b)
            out_wait()
            combine(i, b)
        return carry

    lax.fori_loop(0, N_CHUNKS // 2, step, 0, unroll=False)

    pos_wait()
    gather_drain(0)
    out_wait()
    out_wait()


@jax.jit
def _voxel_sample(xs, ys, zs, table, biasv):
    mesh = plsc.VectorSubcoreMesh(
        core_axis_name="c", subcore_axis_name="s",
        num_cores=NC, num_subcores=NS)
    f = pl.kernel(
        functools.partial(_sc_body),
        out_type=jax.ShapeDtypeStruct((NPTS * 4,), jnp.float32),
        mesh=mesh,
        scratch_types=[
            pltpu.VMEM((2, 3, CP), jnp.float32),
            pltpu.VMEM((2, 8 * CP), jnp.int32),
            pltpu.VMEM((2, 8 * CP), jnp.float32),
            pltpu.VMEM((2, 2 * CP), jnp.int32),
            pltpu.VMEM((2, 8 * CP, 8), jnp.float32),
            pltpu.VMEM((2, CP * 4), jnp.float32),
            pltpu.VMEM((CP * 4,), jnp.float32),
            pltpu.VMEM((LANES,), jnp.float32),
            pltpu.SemaphoreType.DMA,
            pltpu.SemaphoreType.DMA,
            pltpu.SemaphoreType.DMA,
        ],
        compiler_params=pltpu.CompilerParams(
            needs_layout_passes=False, use_tc_tiling_on_sc=False),
    )
    return f(xs, ys, zs, table, biasv)


def kernel(positions, voxels, bias):
    table = _build_table(voxels).reshape(NVOX // 2, 8)
    biasv = jnp.tile(bias[0], 4)
    flat_out = _voxel_sample(positions[:, 0], positions[:, 1],
                             positions[:, 2], table, biasv)
    return flat_out.reshape(NPTS, 4)


# speedup vs baseline: 1.5250x; 1.0058x over previous
import functools

import jax
import jax.numpy as jnp
from jax import lax
from jax.experimental import pallas as pl
from jax.experimental.pallas import tpu as pltpu
from jax.experimental.pallas import tpu_sc as plsc

SIDE = 256
SCALE = 3.0
NPTS = 1048576
NVOX = SIDE * SIDE * SIDE

NC = 2
NS = 16
NW = NC * NS
LANES = 16

CP = 128
PTS_PER_TILE = NPTS // NW
N_CHUNKS = PTS_PER_TILE // CP

ZY = SIDE * SIDE
ZY_PER_TILE = ZY // NW
BROWS = 16
ROWF = 4 * SIDE


def _build_body(src, table, in_v, out_v, dummy_v, pat_s, sem_in, sem_out):
    wid = lax.axis_index("s") * NC + lax.axis_index("c")
    row_base = wid * ZY_PER_TILE
    n_bchunks = ZY_PER_TILE // BROWS

    iota = lax.iota(jnp.int32, LANES)
    cpat = lax.bitwise_and(lax.shift_right_logical(iota, 1), 3)
    xpat = (lax.shift_left(lax.shift_right_logical(iota, 3), 1)
            + lax.bitwise_and(iota, 1))
    pat_s[pl.ds(0, LANES)] = cpat
    pat_s[pl.ds(LANES, LANES)] = xpat

    def in_start(i, b):
        r0 = row_base + i * BROWS
        z = r0 // SIDE
        y0 = lax.rem(r0, SIDE)
        pltpu.async_copy(src.at[0, :, z, pl.ds(y0, BROWS), :],
                         in_v.at[b], sem_in)

    def in_wait():
        pltpu.make_async_copy(
            src.at[0, :, 0, pl.ds(0, BROWS), :], in_v.at[0], sem_in).wait()

    def out_wait():
        pltpu.make_async_copy(
            out_v.at[0], table.at[pl.ds(0, BROWS * ROWF)], sem_out).wait()

    def interleave(i, b):
        cp = pat_s[pl.ds(0, LANES)]
        xp = pat_s[pl.ds(LANES, LANES)]
        for r in range(BROWS):
            yi = jnp.full((LANES,), r, jnp.int32)
            for j in range(ROWF // LANES):
                out_v[b, pl.ds(r * ROWF + j * LANES, LANES)] = (
                    plsc.load_gather(in_v.at[b], [cp, yi, xp + 4 * j]))
        r0 = row_base + i * BROWS
        pltpu.async_copy(out_v.at[b],
                         table.at[pl.ds(r0 * ROWF, BROWS * ROWF)], sem_out)

    in_start(0, 0)
    pltpu.async_copy(table.at[pl.ds(0, BROWS * ROWF)], dummy_v, sem_out)
    pltpu.async_copy(table.at[pl.ds(0, BROWS * ROWF)], dummy_v, sem_out)
    in_start(1, 1)

    def step(k, carry):
        for b in range(2):
            i = 2 * k + b
            in_wait()
            out_wait()
            interleave(i, b)
            in_start(jnp.minimum(i + 2, n_bchunks - 1), b)
        return carry

    lax.fori_loop(0, n_bchunks // 2, step, 0, unroll=False)

    in_wait()
    in_wait()
    out_wait()
    out_wait()


def _build_table(voxels):
    mesh = plsc.VectorSubcoreMesh(
        core_axis_name="c", subcore_axis_name="s",
        num_cores=NC, num_subcores=NS)
    f = pl.kernel(
        _build_body,
        out_type=jax.ShapeDtypeStruct((NVOX * 4,), jnp.float32),
        mesh=mesh,
        scratch_types=[
            pltpu.VMEM((2, 4, BROWS, SIDE), jnp.float32),
            pltpu.VMEM((2, BROWS * ROWF), jnp.float32),
            pltpu.VMEM((BROWS * ROWF,), jnp.float32),
            pltpu.VMEM((2 * LANES,), jnp.int32),
            pltpu.SemaphoreType.DMA,
            pltpu.SemaphoreType.DMA,
        ],
        compiler_params=pltpu.CompilerParams(
            needs_layout_passes=False, use_tc_tiling_on_sc=False),
    )
    return f(voxels)


def _sc_body(xs, ys, zs, table, biasv, out,
             pos_b, idx_b, w_b, par_b, dst_b, outs_b, dummy_v, bias_v,
             sem_pos, sem_g, sem_out):
    wid = lax.axis_index("s") * NC + lax.axis_index("c")
    tile_base = wid * PTS_PER_TILE

    pltpu.sync_copy(biasv, bias_v)
    iota = lax.iota(jnp.int32, LANES)
    colsel2 = lax.rem(iota, 4) * 2
    rowexp0 = lax.div(iota, 4)
    bvec = bias_v[...]

    def pos_start(i, b):
        base = tile_base + i * CP
        pltpu.async_copy(xs.at[pl.ds(base, CP)], pos_b.at[b, 0], sem_pos)
        pltpu.async_copy(ys.at[pl.ds(base, CP)], pos_b.at[b, 1], sem_pos)
        pltpu.async_copy(zs.at[pl.ds(base, CP)], pos_b.at[b, 2], sem_pos)

    def pos_wait():
        for a in range(3):
            pltpu.make_async_copy(
                xs.at[pl.ds(tile_base, CP)], pos_b.at[0, a], sem_pos).wait()

    def compute(b):
        for g in range(CP // LANES):
            sl = pl.ds(g * LANES, LANES)
            x = pos_b[b, 0, sl]
            y = pos_b[b, 1, sl]
            z = pos_b[b, 2, sl]

            def axis(c):
                r = c / SCALE
                i_f = ((r + 1.0) * SIDE - 1.0) * 0.5
                i_f = jnp.minimum(jnp.maximum(i_f, 0.0), float(SIDE - 1))
                i0 = i_f.astype(jnp.int32)
                t = i_f - i0.astype(jnp.float32)
                return i0, t

            x0, tx = axis(x)
            y0, ty = axis(y)
            z0, tz = axis(z)
            dx = jnp.where(x0 < SIDE - 1, 1, 0)
            x1 = x0 + dx
            xh0 = lax.shift_right_logical(x0, 1)
            xh1 = lax.shift_right_logical(x1, 1)
            par0 = lax.bitwise_and(x0, 1)
            par1 = lax.bitwise_and(x1, 1)
            dy2 = jnp.where(y0 < SIDE - 1, SIDE // 2, 0)
            dz2 = jnp.where(z0 < SIDE - 1, (SIDE * SIDE) // 2, 0)
            zyb = (z0 * SIDE + y0) * (SIDE // 2)
            wx0 = 1.0 - tx
            wy0 = 1.0 - ty
            wz0 = 1.0 - tz

            zybs = (zyb, zyb + dy2, zyb + dz2, zyb + dz2 + dy2)
            azys = (wz0 * wy0, wz0 * ty, tz * wy0, tz * ty)
            off = g * LANES
            par_b[b, pl.ds(off, LANES)] = par0
            par_b[b, pl.ds(CP + off, LANES)] = par1
            for zy in range(4):
                for side in range(2):
                    gi = zy * 2 + side
                    idx_b[b, pl.ds(gi * CP + off, LANES)] = (
                        zybs[zy] + (xh1 if side else xh0))
                    w_b[b, pl.ds(gi * CP + off, LANES)] = (
                        azys[zy] * (tx if side else wx0))

    def fire(b):
        for gi in range(8):
            pltpu.async_copy(
                table.at[idx_b.at[b, pl.ds(gi * CP, CP)]],
                dst_b.at[b, pl.ds(gi * CP, CP)],
                sem_g,
            )

    def gather_drain(b):
        pltpu.make_async_copy(
            table.at[pl.ds(0, 8 * CP)], dst_b.at[b], sem_g).wait()

    def out_wait():
        pltpu.make_async_copy(
            outs_b.at[0], out.at[pl.ds(tile_base * 4, CP * 4)],
            sem_out).wait()

    def combine(i, b):
        for j in range(CP // 4):
            rowe = rowexp0 + (4 * j)
            col0 = colsel2 + plsc.load_gather(par_b.at[b], [rowe])
            col1 = colsel2 + plsc.load_gather(par_b.at[b], [rowe + CP])
            acc = bvec
            for zy in range(4):
                for side in range(2):
                    gi = zy * 2 + side
                    r = rowe + (gi * CP)
                    w = plsc.load_gather(w_b.at[b], [r])
                    v = plsc.load_gather(
                        dst_b.at[b], [r, col1 if side else col0])
                    acc = acc + w * v
            outs_b[b, pl.ds(j * LANES, LANES)] = acc
        base = tile_base + i * CP
        pltpu.async_copy(outs_b.at[b],
                         out.at[pl.ds(base * 4, CP * 4)], sem_out)

    pos_start(0, 0)
    pltpu.async_copy(out.at[pl.ds(tile_base * 4, CP * 4)], dummy_v, sem_out)
    pltpu.async_copy(out.at[pl.ds(tile_base * 4, CP * 4)], dummy_v, sem_out)
    pos_wait()
    compute(0)
    fire(0)
    pos_start(1, 1)

    def step(k, carry):
        for b in range(2):
            i = 2 * k + b
            pos_wait()
            compute(1 - b)
            gather_drain(b)
            fire(1 - b)
            pos_start(jnp.minimum(i + 2, N_CHUNKS - 1), b)
            out_wait()
            combine(i, b)
        return carry

    lax.fori_loop(0, N_CHUNKS // 2, step, 0, unroll=False)

    pos_wait()
    gather_drain(0)
    out_wait()
    out_wait()


@jax.jit
def _voxel_sample(xs, ys, zs, table, biasv):
    mesh = plsc.VectorSubcoreMesh(
        core_axis_name="c", subcore_axis_name="s",
        num_cores=NC, num_subcores=NS)
    f = pl.kernel(
        functools.partial(_sc_body),
        out_type=jax.ShapeDtypeStruct((NPTS * 4,), jnp.float32),
        mesh=mesh,
        scratch_types=[
            pltpu.VMEM((2, 3, CP), jnp.float32),
            pltpu.VMEM((2, 8 * CP), jnp.int32),
            pltpu.VMEM((2, 8 * CP), jnp.float32),
            pltpu.VMEM((2, 2 * CP), jnp.int32),
            pltpu.VMEM((2, 8 * CP, 8), jnp.float32),
            pltpu.VMEM((2, CP * 4), jnp.float32),
            pltpu.VMEM((CP * 4,), jnp.float32),
            pltpu.VMEM((LANES,), jnp.float32),
            pltpu.SemaphoreType.DMA,
            pltpu.SemaphoreType.DMA,
            pltpu.SemaphoreType.DMA,
        ],
        compiler_params=pltpu.CompilerParams(
            needs_layout_passes=False, use_tc_tiling_on_sc=False),
    )
    return f(xs, ys, zs, table, biasv)


def kernel(positions, voxels, bias):
    table = _build_table(voxels).reshape(NVOX // 2, 8)
    biasv = jnp.tile(bias[0], 4)
    flat_out = _voxel_sample(positions[:, 0], positions[:, 1],
                             positions[:, 2], table, biasv)
    return flat_out.reshape(NPTS, 4)


# speedup vs baseline: 1.6052x; 1.0526x over previous
import functools

import jax
import jax.numpy as jnp
from jax import lax
from jax.experimental import pallas as pl
from jax.experimental.pallas import tpu as pltpu
from jax.experimental.pallas import tpu_sc as plsc

SIDE = 256
SCALE = 3.0
NPTS = 1048576
NVOX = SIDE * SIDE * SIDE

NC = 2
NS = 16
NW = NC * NS
LANES = 16

CP = 128
PTS_PER_TILE = NPTS // NW
N_CHUNKS = PTS_PER_TILE // CP

ZY = SIDE * SIDE
ZY_PER_TILE = ZY // NW
BROWS = 16
ROWF = 4 * SIDE


def _build_body(src, table, in_v, out_v, dummy_v, pat_s, sem_in, sem_out):
    wid = lax.axis_index("s") * NC + lax.axis_index("c")
    row_base = wid * ZY_PER_TILE
    n_bchunks = ZY_PER_TILE // BROWS

    iota = lax.iota(jnp.int32, LANES)
    cpat = lax.bitwise_and(lax.shift_right_logical(iota, 1), 3)
    xpat = (lax.shift_left(lax.shift_right_logical(iota, 3), 1)
            + lax.bitwise_and(iota, 1))
    pat_s[pl.ds(0, LANES)] = cpat
    pat_s[pl.ds(LANES, LANES)] = xpat

    def in_start(i, b):
        r0 = row_base + i * BROWS
        z = r0 // SIDE
        y0 = lax.rem(r0, SIDE)
        for c in range(4):
            pltpu.async_copy(src.at[0, c, z, pl.ds(y0, BROWS), :],
                             in_v.at[b, :, c, pl.ds(0, SIDE)], sem_in)

    def in_wait():
        for c in range(4):
            pltpu.make_async_copy(
                src.at[0, c, 0, pl.ds(0, BROWS), :],
                in_v.at[0, :, c, pl.ds(0, SIDE)], sem_in).wait()

    def out_wait():
        pltpu.make_async_copy(
            out_v.at[0], table.at[pl.ds(0, BROWS * ROWF)], sem_out).wait()

    def interleave(i, b):
        cp = pat_s[pl.ds(0, LANES)]
        xp = pat_s[pl.ds(LANES, LANES)]
        for r in range(BROWS):
            yi = jnp.full((LANES,), r, jnp.int32)
            for j in range(ROWF // LANES):
                out_v[b, pl.ds(r * ROWF + j * LANES, LANES)] = (
                    plsc.load_gather(in_v.at[b], [yi, cp, xp + 4 * j]))
        r0 = row_base + i * BROWS
        pltpu.async_copy(out_v.at[b],
                         table.at[pl.ds(r0 * ROWF, BROWS * ROWF)], sem_out)

    in_start(0, 0)
    pltpu.async_copy(table.at[pl.ds(0, BROWS * ROWF)], dummy_v, sem_out)
    pltpu.async_copy(table.at[pl.ds(0, BROWS * ROWF)], dummy_v, sem_out)
    in_start(1, 1)

    def step(k, carry):
        for b in range(2):
            i = 2 * k + b
            in_wait()
            out_wait()
            interleave(i, b)
            in_start(jnp.minimum(i + 2, n_bchunks - 1), b)
        return carry

    lax.fori_loop(0, n_bchunks // 2, step, 0, unroll=False)

    in_wait()
    in_wait()
    out_wait()
    out_wait()


def _build_table(voxels):
    mesh = plsc.VectorSubcoreMesh(
        core_axis_name="c", subcore_axis_name="s",
        num_cores=NC, num_subcores=NS)
    f = pl.kernel(
        _build_body,
        out_type=jax.ShapeDtypeStruct((NVOX * 4,), jnp.float32),
        mesh=mesh,
        scratch_types=[
            pltpu.VMEM((2, BROWS, 4, SIDE + 4), jnp.float32),
            pltpu.VMEM((2, BROWS * ROWF), jnp.float32),
            pltpu.VMEM((BROWS * ROWF,), jnp.float32),
            pltpu.VMEM((2 * LANES,), jnp.int32),
            pltpu.SemaphoreType.DMA,
            pltpu.SemaphoreType.DMA,
        ],
        compiler_params=pltpu.CompilerParams(
            needs_layout_passes=False, use_tc_tiling_on_sc=False),
    )
    return f(voxels)


def _sc_body(xs, ys, zs, table, biasv, out,
             pos_b, idx_b, w_b, par_b, dst_b, outs_b, dummy_v, bias_v,
             sem_pos, sem_g, sem_out):
    wid = lax.axis_index("s") * NC + lax.axis_index("c")
    tile_base = wid * PTS_PER_TILE

    pltpu.sync_copy(biasv, bias_v)
    iota = lax.iota(jnp.int32, LANES)
    colsel2 = lax.rem(iota, 4) * 2
    rowexp0 = lax.div(iota, 4)
    bvec = bias_v[...]

    def pos_start(i, b):
        base = tile_base + i * CP
        pltpu.async_copy(xs.at[pl.ds(base, CP)], pos_b.at[b, 0], sem_pos)
        pltpu.async_copy(ys.at[pl.ds(base, CP)], pos_b.at[b, 1], sem_pos)
        pltpu.async_copy(zs.at[pl.ds(base, CP)], pos_b.at[b, 2], sem_pos)

    def pos_wait():
        for a in range(3):
            pltpu.make_async_copy(
                xs.at[pl.ds(tile_base, CP)], pos_b.at[0, a], sem_pos).wait()

    def compute(b):
        for g in range(CP // LANES):
            sl = pl.ds(g * LANES, LANES)
            x = pos_b[b, 0, sl]
            y = pos_b[b, 1, sl]
            z = pos_b[b, 2, sl]

            def axis(c):
                r = c / SCALE
                i_f = ((r + 1.0) * SIDE - 1.0) * 0.5
                i_f = jnp.minimum(jnp.maximum(i_f, 0.0), float(SIDE - 1))
                i0 = i_f.astype(jnp.int32)
                t = i_f - i0.astype(jnp.float32)
                return i0, t

            x0, tx = axis(x)
            y0, ty = axis(y)
            z0, tz = axis(z)
            dx = jnp.where(x0 < SIDE - 1, 1, 0)
            x1 = x0 + dx
            xh0 = lax.shift_right_logical(x0, 1)
            xh1 = lax.shift_right_logical(x1, 1)
            par0 = lax.bitwise_and(x0, 1)
            par1 = lax.bitwise_and(x1, 1)
            dy2 = jnp.where(y0 < SIDE - 1, SIDE // 2, 0)
            dz2 = jnp.where(z0 < SIDE - 1, (SIDE * SIDE) // 2, 0)
            zyb = (z0 * SIDE + y0) * (SIDE // 2)
            wx0 = 1.0 - tx
            wy0 = 1.0 - ty
            wz0 = 1.0 - tz

            zybs = (zyb, zyb + dy2, zyb + dz2, zyb + dz2 + dy2)
            azys = (wz0 * wy0, wz0 * ty, tz * wy0, tz * ty)
            off = g * LANES
            par_b[b, pl.ds(off, LANES)] = par0
            par_b[b, pl.ds(CP + off, LANES)] = par1
            for zy in range(4):
                for side in range(2):
                    gi = zy * 2 + side
                    idx_b[b, pl.ds(gi * CP + off, LANES)] = (
                        zybs[zy] + (xh1 if side else xh0))
                    w_b[b, pl.ds(gi * CP + off, LANES)] = (
                        azys[zy] * (tx if side else wx0))

    def fire(b):
        for gi in range(8):
            pltpu.async_copy(
                table.at[idx_b.at[b, pl.ds(gi * CP, CP)]],
                dst_b.at[b, pl.ds(gi * CP, CP)],
                sem_g,
            )

    def gather_drain(b):
        pltpu.make_async_copy(
            table.at[pl.ds(0, 8 * CP)], dst_b.at[b], sem_g).wait()

    def out_wait():
        pltpu.make_async_copy(
            outs_b.at[0], out.at[pl.ds(tile_base * 4, CP * 4)],
            sem_out).wait()

    def combine(i, b):
        for j in range(CP // 4):
            rowe = rowexp0 + (4 * j)
            col0 = colsel2 + plsc.load_gather(par_b.at[b], [rowe])
            col1 = colsel2 + plsc.load_gather(par_b.at[b], [rowe + CP])
            acc = bvec
            for zy in range(4):
                for side in range(2):
                    gi = zy * 2 + side
                    r = rowe + (gi * CP)
                    w = plsc.load_gather(w_b.at[b], [r])
                    v = plsc.load_gather(
                        dst_b.at[b], [r, col1 if side else col0])
                    acc = acc + w * v
            outs_b[b, pl.ds(j * LANES, LANES)] = acc
        base = tile_base + i * CP
        pltpu.async_copy(outs_b.at[b],
                         out.at[pl.ds(base * 4, CP * 4)], sem_out)

    pos_start(0, 0)
    pltpu.async_copy(out.at[pl.ds(tile_base * 4, CP * 4)], dummy_v, sem_out)
    pltpu.async_copy(out.at[pl.ds(tile_base * 4, CP * 4)], dummy_v, sem_out)
    pos_wait()
    compute(0)
    fire(0)
    pos_start(1, 1)

    def step(k, carry):
        for b in range(2):
            i = 2 * k + b
            pos_wait()
            compute(1 - b)
            gather_drain(b)
            fire(1 - b)
            pos_start(jnp.minimum(i + 2, N_CHUNKS - 1), b)
            out_wait()
            combine(i, b)
        return carry

    lax.fori_loop(0, N_CHUNKS // 2, step, 0, unroll=False)

    pos_wait()
    gather_drain(0)
    out_wait()
    out_wait()


@jax.jit
def _voxel_sample(xs, ys, zs, table, biasv):
    mesh = plsc.VectorSubcoreMesh(
        core_axis_name="c", subcore_axis_name="s",
        num_cores=NC, num_subcores=NS)
    f = pl.kernel(
        functools.partial(_sc_body),
        out_type=jax.ShapeDtypeStruct((NPTS * 4,), jnp.float32),
        mesh=mesh,
        scratch_types=[
            pltpu.VMEM((2, 3, CP), jnp.float32),
            pltpu.VMEM((2, 8 * CP), jnp.int32),
            pltpu.VMEM((2, 8 * CP), jnp.float32),
            pltpu.VMEM((2, 2 * CP), jnp.int32),
            pltpu.VMEM((2, 8 * CP, 8), jnp.float32),
            pltpu.VMEM((2, CP * 4), jnp.float32),
            pltpu.VMEM((CP * 4,), jnp.float32),
            pltpu.VMEM((LANES,), jnp.float32),
            pltpu.SemaphoreType.DMA,
            pltpu.SemaphoreType.DMA,
            pltpu.SemaphoreType.DMA,
        ],
        compiler_params=pltpu.CompilerParams(
            needs_layout_passes=False, use_tc_tiling_on_sc=False),
    )
    return f(xs, ys, zs, table, biasv)


def kernel(positions, voxels, bias):
    table = _build_table(voxels).reshape(NVOX // 2, 8)
    biasv = jnp.tile(bias[0], 4)
    flat_out = _voxel_sample(positions[:, 0], positions[:, 1],
                             positions[:, 2], table, biasv)
    return flat_out.reshape(NPTS, 4)
